# Initial kernel scaffold; baseline (speedup 1.0000x reference)
import jax
import jax.numpy as jnp
from jax.experimental import pallas as pl


def kernel(output, bank, update):
    raise NotImplementedError("write your pallas kernel here")


# speedup vs baseline: 1.2808x; 1.2808x over previous
import jax
import jax.numpy as jnp
from jax.experimental import pallas as pl
from jax.experimental.pallas import tpu as pltpu


def _mb_kernel(nb_out, u_ref, out_in_ref, bank_ref, outbank_ref, newbank_ref):
    i = pl.program_id(0)
    blk = bank_ref[...]
    outbank_ref[...] = blk.T
    upd = (u_ref[0] != 0) & (i < nb_out)
    newbank_ref[...] = jnp.where(upd, out_in_ref[...], blk)


def kernel(output, bank, update):
    size, dim = bank.shape
    batch = output.shape[0]
    r = 2048
    nb_out = batch // r
    grid = size // r
    u = jnp.asarray(update, jnp.int32).reshape(1)

    import functools
    body = functools.partial(_mb_kernel, nb_out)
    out_bank, new_bank = pl.pallas_call(
        body,
        grid=(grid,),
        in_specs=[
            pl.BlockSpec(memory_space=pltpu.SMEM),
            pl.BlockSpec((r, dim), lambda i: (jnp.minimum(i, nb_out - 1), 0)),
            pl.BlockSpec((r, dim), lambda i: (i, 0)),
        ],
        out_specs=[
            pl.BlockSpec((dim, r), lambda i: (0, i)),
            pl.BlockSpec((r, dim), lambda i: (i, 0)),
        ],
        out_shape=[
            jax.ShapeDtypeStruct((dim, size), bank.dtype),
            jax.ShapeDtypeStruct((size, dim), bank.dtype),
        ],
    )(u, output, bank)
    return (output, out_bank, new_bank)
